# baseline (device time: 10440 ns/iter reference)
import jax
import jax.numpy as jnp
from jax import lax
from jax.experimental import pallas as pl
from jax.experimental.pallas import tpu as pltpu

N_DEV = 4
N_CHUNK = 4


def kernel(x):
    m, n = x.shape
    cm = m // N_CHUNK

    def body(x_ref, out_ref, stats_ref, send_sems, recv_sems):
        my = lax.axis_index("i")

        barrier_sem = pltpu.get_barrier_semaphore()
        for r in range(1, N_DEV):
            pl.semaphore_signal(
                barrier_sem, inc=1,
                device_id=((my + r) % N_DEV,),
                device_id_type=pl.DeviceIdType.MESH,
            )

        rdmas = [[None] * N_DEV for _ in range(N_CHUNK)]

        for c in range(N_CHUNK):
            rows = pl.ds(c * cm, cm)
            e = jnp.exp(x_ref[rows, :])
            out_ref[rows, :] = e
            ls = jnp.sum(e, axis=1, keepdims=True)
            stats_ref[c, 0, :, :] = jnp.transpose(ls, (1, 0))
            if c == 0:
                pl.semaphore_wait(barrier_sem, N_DEV - 1)
            for r in range(1, N_DEV):
                rdma = pltpu.make_async_remote_copy(
                    src_ref=stats_ref.at[c, 0],
                    dst_ref=stats_ref.at[c, N_DEV - r],
                    send_sem=send_sems.at[c, r],
                    recv_sem=recv_sems.at[c, N_DEV - r],
                    device_id=((my + r) % N_DEV,),
                    device_id_type=pl.DeviceIdType.MESH,
                )
                rdma.start()
                rdmas[c][r] = rdma

        for c in range(N_CHUNK):
            for r in range(1, N_DEV):
                rdmas[c][r].wait()
            gs = jnp.sum(stats_ref[c, :, :, :], axis=0)
            scale_row = 1.0 / gs
            rows = pl.ds(c * cm, cm)
            out_ref[rows, :] = out_ref[rows, :] * jnp.transpose(scale_row, (1, 0))

    return pl.pallas_call(
        body,
        out_shape=jax.ShapeDtypeStruct((m, n), x.dtype),
        in_specs=[pl.BlockSpec(memory_space=pltpu.VMEM)],
        out_specs=pl.BlockSpec(memory_space=pltpu.VMEM),
        scratch_shapes=[
            pltpu.VMEM((N_CHUNK, N_DEV, 1, cm), jnp.float32),
            pltpu.SemaphoreType.DMA((N_CHUNK, N_DEV)),
            pltpu.SemaphoreType.DMA((N_CHUNK, N_DEV)),
        ],
        compiler_params=pltpu.CompilerParams(collective_id=0),
    )(x)
